# Initial kernel scaffold; baseline (speedup 1.0000x reference)
import jax
import jax.numpy as jnp
from jax.experimental import pallas as pl


def kernel(positions, indeces_i, indeces_j, offsets):
    raise NotImplementedError("write your pallas kernel here")


# speedup vs baseline: 4.0821x; 4.0821x over previous
import functools

import jax
import jax.numpy as jnp
from jax import lax
from jax.experimental import pallas as pl
from jax.experimental.pallas import tpu as pltpu, tpu_sc as plsc

N_NODES = 100000
N_EDGES = 6400000
ROW = 8

NUM_CORES = 2
NUM_SUBCORES = 16
LANES = 16
NW = NUM_CORES * NUM_SUBCORES

E_PER_W = N_EDGES // NW
CHUNK = 2000
N_CHUNKS = E_PER_W // CHUNK
FLAT = 3 * CHUNK


def _edge_kernel(pos_hbm, ii_hbm, ij_hbm, off_hbm, out_hbm,
                 tab_s, ii_v, ij_v, gi_v, gj_v, ob_v, sem):
    sid = lax.axis_index("s")
    wid = sid * NUM_CORES + lax.axis_index("c")
    w_base = wid * E_PER_W

    @pl.when(sid == 0)
    def _():
        pltpu.sync_copy(pos_hbm, tab_s)

    plsc.subcore_barrier()

    def chunk_body(g, carry):
        base = w_base + g * CHUNK
        pltpu.sync_copy(ii_hbm.at[pl.ds(base, CHUNK)], ii_v)
        pltpu.sync_copy(ij_hbm.at[pl.ds(base, CHUNK)], ij_v)
        pltpu.sync_copy(off_hbm.at[pl.ds(3 * base, FLAT)], ob_v)
        cp_i = pltpu.make_async_copy(tab_s.at[ii_v], gi_v, sem)
        cp_i.start()
        cp_j = pltpu.make_async_copy(tab_s.at[ij_v], gj_v, sem)
        cp_j.start()
        cp_i.wait()
        cp_j.wait()

        def vec_body(t, carry2):
            f0 = t * LANES
            f = f0 + lax.iota(jnp.int32, LANES)
            row = (f * 21846) >> 16
            col = f - 3 * row
            pi = plsc.load_gather(gi_v, [row, col])
            pj = plsc.load_gather(gj_v, [row, col])
            off = ob_v[pl.ds(f0, LANES)]
            ob_v[pl.ds(f0, LANES)] = off + pj - pi
            return carry2

        lax.fori_loop(0, FLAT // LANES, vec_body, 0, unroll=4)
        pltpu.sync_copy(ob_v, out_hbm.at[pl.ds(3 * base, FLAT)])
        return carry

    lax.fori_loop(0, N_CHUNKS, chunk_body, 0)


@jax.jit
def kernel(positions, indeces_i, indeces_j, offsets):
    mesh = plsc.VectorSubcoreMesh(core_axis_name="c", subcore_axis_name="s")
    run = pl.kernel(
        _edge_kernel,
        out_type=jax.ShapeDtypeStruct((3 * N_EDGES,), jnp.float32),
        mesh=mesh,
        compiler_params=pltpu.CompilerParams(
            needs_layout_passes=False, use_tc_tiling_on_sc=False),
        scratch_types=[
            pltpu.VMEM_SHARED((N_NODES, ROW), jnp.float32),
            pltpu.VMEM((CHUNK,), jnp.int32),
            pltpu.VMEM((CHUNK,), jnp.int32),
            pltpu.VMEM((CHUNK, ROW), jnp.float32),
            pltpu.VMEM((CHUNK, ROW), jnp.float32),
            pltpu.VMEM((FLAT,), jnp.float32),
            pltpu.SemaphoreType.DMA,
        ],
    )
    pos_pad = jnp.pad(positions, ((0, 0), (0, ROW - 3)))
    out_flat = run(
        pos_pad,
        indeces_i.astype(jnp.int32),
        indeces_j.astype(jnp.int32),
        offsets.reshape(3 * N_EDGES),
    )
    return out_flat.reshape(N_EDGES, 3)


# speedup vs baseline: 47.3823x; 11.6073x over previous
import functools

import jax
import jax.numpy as jnp
from jax import lax
from jax.experimental import pallas as pl
from jax.experimental.pallas import tpu as pltpu, tpu_sc as plsc

N_NODES = 100000
N_EDGES = 6400000
ROW = 8

NUM_CORES = 2
NUM_SUBCORES = 16
LANES = 16
NW = NUM_CORES * NUM_SUBCORES

E_PER_W = N_EDGES // NW
CHUNK = 2000
N_CHUNKS = E_PER_W // CHUNK


def _edge_kernel(pos_hbm, ii_hbm, ij_hbm, o0_hbm, o1_hbm, o2_hbm,
                 u0_hbm, u1_hbm, u2_hbm,
                 tab_s, ii_v, ij_v, gi_v, gj_v, b0_v, b1_v, b2_v, sem):
    sid = lax.axis_index("s")
    wid = sid * NUM_CORES + lax.axis_index("c")
    w_base = wid * E_PER_W

    @pl.when(sid == 0)
    def _():
        pltpu.sync_copy(pos_hbm, tab_s)

    plsc.subcore_barrier()

    obufs = (b0_v, b1_v, b2_v)

    def chunk_body(g, carry):
        base = w_base + g * CHUNK
        pltpu.sync_copy(ii_hbm.at[pl.ds(base, CHUNK)], ii_v)
        pltpu.sync_copy(ij_hbm.at[pl.ds(base, CHUNK)], ij_v)
        pltpu.sync_copy(o0_hbm.at[pl.ds(base, CHUNK)], b0_v)
        pltpu.sync_copy(o1_hbm.at[pl.ds(base, CHUNK)], b1_v)
        pltpu.sync_copy(o2_hbm.at[pl.ds(base, CHUNK)], b2_v)
        cp_i = pltpu.make_async_copy(tab_s.at[ii_v], gi_v, sem)
        cp_i.start()
        cp_j = pltpu.make_async_copy(tab_s.at[ij_v], gj_v, sem)
        cp_j.start()
        cp_i.wait()
        cp_j.wait()

        def vec_body(t, carry2):
            e0 = t * LANES
            ev = e0 + lax.iota(jnp.int32, LANES)
            for k in range(3):
                ck = jnp.full((LANES,), k, jnp.int32)
                pi = plsc.load_gather(gi_v, [ev, ck])
                pj = plsc.load_gather(gj_v, [ev, ck])
                ob = obufs[k]
                ob[pl.ds(e0, LANES)] = ob[pl.ds(e0, LANES)] + pj - pi
            return carry2

        lax.fori_loop(0, CHUNK // LANES, vec_body, 0, unroll=4)
        pltpu.sync_copy(b0_v, u0_hbm.at[pl.ds(base, CHUNK)])
        pltpu.sync_copy(b1_v, u1_hbm.at[pl.ds(base, CHUNK)])
        pltpu.sync_copy(b2_v, u2_hbm.at[pl.ds(base, CHUNK)])
        return carry

    lax.fori_loop(0, N_CHUNKS, chunk_body, 0)


@jax.jit
def kernel(positions, indeces_i, indeces_j, offsets):
    mesh = plsc.VectorSubcoreMesh(core_axis_name="c", subcore_axis_name="s")
    vec = jax.ShapeDtypeStruct((N_EDGES,), jnp.float32)
    run = pl.kernel(
        _edge_kernel,
        out_type=(vec, vec, vec),
        mesh=mesh,
        compiler_params=pltpu.CompilerParams(
            needs_layout_passes=False, use_tc_tiling_on_sc=False),
        scratch_types=[
            pltpu.VMEM_SHARED((N_NODES, ROW), jnp.float32),
            pltpu.VMEM((CHUNK,), jnp.int32),
            pltpu.VMEM((CHUNK,), jnp.int32),
            pltpu.VMEM((CHUNK, ROW), jnp.float32),
            pltpu.VMEM((CHUNK, ROW), jnp.float32),
            pltpu.VMEM((CHUNK,), jnp.float32),
            pltpu.VMEM((CHUNK,), jnp.float32),
            pltpu.VMEM((CHUNK,), jnp.float32),
            pltpu.SemaphoreType.DMA,
        ],
    )
    pos_pad = jnp.pad(positions, ((0, 0), (0, ROW - 3)))
    u0, u1, u2 = run(
        pos_pad,
        indeces_i.astype(jnp.int32),
        indeces_j.astype(jnp.int32),
        offsets[:, 0],
        offsets[:, 1],
        offsets[:, 2],
    )
    return jnp.stack([u0, u1, u2], axis=1)


# speedup vs baseline: 60.5110x; 1.2771x over previous
import functools

import jax
import jax.numpy as jnp
from jax import lax
from jax.experimental import pallas as pl
from jax.experimental.pallas import tpu as pltpu, tpu_sc as plsc

N_NODES = 100000
N_EDGES = 6400000
ROW = 8

NUM_CORES = 2
NUM_SUBCORES = 16
LANES = 16
NW = NUM_CORES * NUM_SUBCORES

E_PER_W = N_EDGES // NW
CHUNK = 2000
N_CHUNKS = E_PER_W // CHUNK


def _edge_kernel(pos_hbm, ii_hbm, ij_hbm, o0_hbm, o1_hbm, o2_hbm,
                 u0_hbm, u1_hbm, u2_hbm,
                 tab_s, ii_v, ij_v, gi_v, gj_v, b0_v, b1_v, b2_v,
                 sem_idx, sem_off, sem_g, sem_out):
    sid = lax.axis_index("s")
    wid = sid * NUM_CORES + lax.axis_index("c")
    w_base = wid * E_PER_W

    @pl.when(sid == 0)
    def _():
        pltpu.sync_copy(pos_hbm, tab_s)

    plsc.subcore_barrier()

    obufs = (b0_v, b1_v, b2_v)

    def chunk_body(g, carry):
        base = w_base + g * CHUNK
        ld = pl.ds(base, CHUNK)
        c_ii = pltpu.make_async_copy(ii_hbm.at[ld], ii_v, sem_idx)
        c_ij = pltpu.make_async_copy(ij_hbm.at[ld], ij_v, sem_idx)
        c_o0 = pltpu.make_async_copy(o0_hbm.at[ld], b0_v, sem_off)
        c_o1 = pltpu.make_async_copy(o1_hbm.at[ld], b1_v, sem_off)
        c_o2 = pltpu.make_async_copy(o2_hbm.at[ld], b2_v, sem_off)
        c_ii.start(); c_ij.start(); c_o0.start(); c_o1.start(); c_o2.start()
        c_ii.wait(); c_ij.wait()
        cp_i = pltpu.make_async_copy(tab_s.at[ii_v], gi_v, sem_g)
        cp_i.start()
        cp_j = pltpu.make_async_copy(tab_s.at[ij_v], gj_v, sem_g)
        cp_j.start()
        c_o0.wait(); c_o1.wait(); c_o2.wait()
        cp_i.wait()
        cp_j.wait()

        def vec_body(t, carry2):
            e0 = t * LANES
            ev = e0 + lax.iota(jnp.int32, LANES)
            for k in range(3):
                ck = jnp.full((LANES,), k, jnp.int32)
                pi = plsc.load_gather(gi_v, [ev, ck])
                pj = plsc.load_gather(gj_v, [ev, ck])
                ob = obufs[k]
                ob[pl.ds(e0, LANES)] = ob[pl.ds(e0, LANES)] + pj - pi
            return carry2

        lax.fori_loop(0, CHUNK // LANES, vec_body, 0, unroll=4)
        s0 = pltpu.make_async_copy(b0_v, u0_hbm.at[ld], sem_out)
        s1 = pltpu.make_async_copy(b1_v, u1_hbm.at[ld], sem_out)
        s2 = pltpu.make_async_copy(b2_v, u2_hbm.at[ld], sem_out)
        s0.start(); s1.start(); s2.start()
        s0.wait(); s1.wait(); s2.wait()
        return carry

    lax.fori_loop(0, N_CHUNKS, chunk_body, 0)


@jax.jit
def kernel(positions, indeces_i, indeces_j, offsets):
    mesh = plsc.VectorSubcoreMesh(core_axis_name="c", subcore_axis_name="s")
    vec = jax.ShapeDtypeStruct((N_EDGES,), jnp.float32)
    run = pl.kernel(
        _edge_kernel,
        out_type=(vec, vec, vec),
        mesh=mesh,
        compiler_params=pltpu.CompilerParams(
            needs_layout_passes=False, use_tc_tiling_on_sc=False),
        scratch_types=[
            pltpu.VMEM_SHARED((N_NODES, ROW), jnp.float32),
            pltpu.VMEM((CHUNK,), jnp.int32),
            pltpu.VMEM((CHUNK,), jnp.int32),
            pltpu.VMEM((CHUNK, ROW), jnp.float32),
            pltpu.VMEM((CHUNK, ROW), jnp.float32),
            pltpu.VMEM((CHUNK,), jnp.float32),
            pltpu.VMEM((CHUNK,), jnp.float32),
            pltpu.VMEM((CHUNK,), jnp.float32),
            pltpu.SemaphoreType.DMA,
            pltpu.SemaphoreType.DMA,
            pltpu.SemaphoreType.DMA,
            pltpu.SemaphoreType.DMA,
        ],
    )
    pos_pad = jnp.pad(positions, ((0, 0), (0, ROW - 3)))
    u0, u1, u2 = run(
        pos_pad,
        indeces_i.astype(jnp.int32),
        indeces_j.astype(jnp.int32),
        offsets[:, 0],
        offsets[:, 1],
        offsets[:, 2],
    )
    return jnp.stack([u0, u1, u2], axis=1)


# speedup vs baseline: 83.3547x; 1.3775x over previous
import functools

import jax
import jax.numpy as jnp
from jax import lax
from jax.experimental import pallas as pl
from jax.experimental.pallas import tpu as pltpu, tpu_sc as plsc

N_NODES = 100000
N_EDGES = 6400000
ROW = 8

NUM_CORES = 2
NUM_SUBCORES = 16
LANES = 16
NW = NUM_CORES * NUM_SUBCORES

E_PER_W = N_EDGES // NW
CHUNK = 1000
N_CHUNKS = E_PER_W // CHUNK
NPHASE = 3
ROUNDS = (N_CHUNKS + NPHASE - 1) // NPHASE

_VEC_SCRATCH = [
    pltpu.VMEM((CHUNK,), jnp.int32),
    pltpu.VMEM((CHUNK,), jnp.int32),
    pltpu.VMEM((CHUNK, ROW), jnp.float32),
    pltpu.VMEM((CHUNK, ROW), jnp.float32),
    pltpu.VMEM((CHUNK,), jnp.float32),
    pltpu.VMEM((CHUNK,), jnp.float32),
    pltpu.VMEM((CHUNK,), jnp.float32),
    pltpu.VMEM((CHUNK,), jnp.float32),
    pltpu.VMEM((CHUNK,), jnp.float32),
    pltpu.VMEM((CHUNK,), jnp.float32),
    pltpu.SemaphoreType.DMA,
    pltpu.SemaphoreType.DMA,
    pltpu.SemaphoreType.DMA,
    pltpu.SemaphoreType.DMA,
]


def _edge_kernel(pos_hbm, ii_hbm, ij_hbm, o0_hbm, o1_hbm, o2_hbm,
                 u0_hbm, u1_hbm, u2_hbm, tab_s, *scratch):
    sid = lax.axis_index("s")
    wid = sid * NUM_CORES + lax.axis_index("c")
    w_base = wid * E_PER_W

    nper = len(_VEC_SCRATCH)
    sets = [scratch[p * nper:(p + 1) * nper] for p in range(NPHASE)]

    @pl.when(sid == 0)
    def _():
        pltpu.sync_copy(pos_hbm, tab_s)

    plsc.subcore_barrier()

    def clamp(c):
        return jnp.minimum(c, N_CHUNKS - 1)

    def in_copies(s, c):
        (ii_v, ij_v, _, _, b0, b1, b2, _, _, _,
         sem_idx, sem_off, _, _) = s
        ld = pl.ds(w_base + clamp(c) * CHUNK, CHUNK)
        return (
            (pltpu.make_async_copy(ii_hbm.at[ld], ii_v, sem_idx),
             pltpu.make_async_copy(ij_hbm.at[ld], ij_v, sem_idx)),
            (pltpu.make_async_copy(o0_hbm.at[ld], b0, sem_off),
             pltpu.make_async_copy(o1_hbm.at[ld], b1, sem_off),
             pltpu.make_async_copy(o2_hbm.at[ld], b2, sem_off)),
        )

    def out_copies(s, c):
        (_, _, _, _, _, _, _, r0, r1, r2, _, _, _, sem_out) = s
        ld = pl.ds(w_base + clamp(c) * CHUNK, CHUNK)
        return (pltpu.make_async_copy(r0, u0_hbm.at[ld], sem_out),
                pltpu.make_async_copy(r1, u1_hbm.at[ld], sem_out),
                pltpu.make_async_copy(r2, u2_hbm.at[ld], sem_out))

    def gather_copies(s):
        (ii_v, ij_v, gi_v, gj_v, _, _, _, _, _, _, _, _, sem_g, _) = s
        return (pltpu.make_async_copy(tab_s.at[ii_v], gi_v, sem_g),
                pltpu.make_async_copy(tab_s.at[ij_v], gj_v, sem_g))

    def start_in(s, c):
        idx, off = in_copies(s, c)
        for cp in idx + off:
            cp.start()

    def wait_idx(s, c):
        for cp in in_copies(s, c)[0]:
            cp.wait()

    def wait_off(s, c):
        for cp in in_copies(s, c)[1]:
            cp.wait()

    def start_gather(s):
        for cp in gather_copies(s):
            cp.start()

    def wait_gather(s):
        for cp in gather_copies(s):
            cp.wait()

    def compute(s):
        (_, _, gi_v, gj_v, b0, b1, b2, r0, r1, r2, _, _, _, _) = s
        bs = (b0, b1, b2)
        rs = (r0, r1, r2)

        def vec_body(t, carry):
            e0 = t * LANES
            ev = e0 + lax.iota(jnp.int32, LANES)
            for k in range(3):
                ck = jnp.full((LANES,), k, jnp.int32)
                pi = plsc.load_gather(gi_v, [ev, ck])
                pj = plsc.load_gather(gj_v, [ev, ck])
                rs[k][pl.ds(e0, LANES)] = bs[k][pl.ds(e0, LANES)] + pj - pi
            return carry

        lax.fori_loop(0, CHUNK // LANES, vec_body, 0, unroll=4)

    def phase(p, s, g3, c):
        wait_gather(s)
        wait_off(s, c)

        @pl.when(g3 > 0)
        def _():
            for cp in out_copies(s, c - NPHASE):
                cp.wait()

        compute(s)
        for cp in out_copies(s, c):
            cp.start()
        start_in(s, c + NPHASE)

    start_in(sets[0], 0)
    start_in(sets[1], 1)
    start_in(sets[2], 2)
    wait_idx(sets[0], 0)
    start_gather(sets[0])

    def round_body(g3, carry):
        cA = g3 * NPHASE
        wait_idx(sets[1], cA + 1)
        start_gather(sets[1])
        phase(0, sets[0], g3, cA)
        wait_idx(sets[2], cA + 2)
        start_gather(sets[2])
        phase(1, sets[1], g3, cA + 1)
        wait_idx(sets[0], cA + 3)
        start_gather(sets[0])
        phase(2, sets[2], g3, cA + 2)
        return carry

    lax.fori_loop(0, ROUNDS, round_body, 0)

    last = ROUNDS * NPHASE
    wait_gather(sets[0])
    wait_off(sets[0], last)
    wait_idx(sets[1], last + 1)
    wait_off(sets[1], last + 1)
    wait_idx(sets[2], last + 2)
    wait_off(sets[2], last + 2)
    for p in range(NPHASE):
        for cp in out_copies(sets[p], last - NPHASE + p):
            cp.wait()


@jax.jit
def kernel(positions, indeces_i, indeces_j, offsets):
    mesh = plsc.VectorSubcoreMesh(core_axis_name="c", subcore_axis_name="s")
    vec = jax.ShapeDtypeStruct((N_EDGES,), jnp.float32)
    run = pl.kernel(
        _edge_kernel,
        out_type=(vec, vec, vec),
        mesh=mesh,
        compiler_params=pltpu.CompilerParams(
            needs_layout_passes=False, use_tc_tiling_on_sc=False),
        scratch_types=(
            [pltpu.VMEM_SHARED((N_NODES, ROW), jnp.float32)]
            + _VEC_SCRATCH * NPHASE
        ),
    )
    pos_pad = jnp.pad(positions, ((0, 0), (0, ROW - 3)))
    u0, u1, u2 = run(
        pos_pad,
        indeces_i.astype(jnp.int32),
        indeces_j.astype(jnp.int32),
        offsets[:, 0],
        offsets[:, 1],
        offsets[:, 2],
    )
    return jnp.stack([u0, u1, u2], axis=1)
